# baseline (speedup 1.0000x reference)
import functools

import jax
import jax.numpy as jnp
from jax import lax
from jax.experimental import pallas as pl
from jax.experimental.pallas import tpu as pltpu
from jax.experimental.pallas import tpu_sc as plsc

_CHANNELS = 16
_SCALAR_MAX = 4
_BASIS = 9
_OUT_W = 2 * _BASIS * _CHANNELS
_NPAIR = 81

_AN_IDX = jnp.array([0, 0, 0, 0, 0, 0, 1, 2, 3], jnp.int32)
_AN_VALID = jnp.array([False, True, False, False, False, False, True, True, True])
_SDIMS = jnp.array([3, 4, 4, 4], jnp.int32)

_C = 128
_LANES = 16


def _build_table(w):
    ia = _AN_IDX[:, None]
    ib = _AN_IDX[None, :]
    valid = _AN_VALID[:, None] & _AN_VALID[None, :]
    sfa = w[ia * 4 + ib].reshape(9, 9, _SCALAR_MAX, _CHANNELS)
    sfb = w[ib * 4 + ia].reshape(9, 9, _SCALAR_MAX, _CHANNELS)
    pad = ((0, 0), (0, 0), (0, _BASIS - _SCALAR_MAX), (0, 0))
    sfa_p = jnp.pad(sfa, pad)
    sfb_p = jnp.pad(sfb, pad)
    rows = jnp.arange(_BASIS)[None, None, :, None]
    ma = valid[:, :, None, None] & (rows < _SDIMS[ia][:, :, None, None])
    mb = valid[:, :, None, None] & (rows < _SDIMS[ib][:, :, None, None])
    ta = jnp.where(ma, sfa_p, 0.0)
    tb = jnp.where(mb, sfb_p, 0.0)
    return jnp.concatenate([ta, tb], axis=-1).reshape(_NPAIR, _OUT_W)


def _sc_kernel(num_workers, n_atoms, e_total):
    epw = e_total // num_workers
    nchunks = pl.cdiv(epw, _C)
    if nchunks % 2 or nchunks < 4:
        raise ValueError("unsupported edge count")
    mesh = plsc.VectorSubcoreMesh(core_axis_name="c", subcore_axis_name="s")

    @functools.partial(
        pl.kernel,
        mesh=mesh,
        compiler_params=pltpu.CompilerParams(
            use_tc_tiling_on_sc=False, needs_layout_passes=False),
        out_type=jax.ShapeDtypeStruct((e_total, _OUT_W), jnp.float32),
        scratch_types=[
            pltpu.VMEM((n_atoms,), jnp.int32),
            pltpu.VMEM((epw,), jnp.int32),
            pltpu.VMEM((epw,), jnp.int32),
            pltpu.VMEM((_NPAIR, _OUT_W), jnp.float32),
            pltpu.VMEM((_C, _OUT_W), jnp.float32),
            pltpu.VMEM((_C, _OUT_W), jnp.float32),
            pltpu.SemaphoreType.DMA,
            pltpu.SemaphoreType.DMA,
        ],
    )
    def body(an_hbm, eidx_hbm, table_hbm, out_hbm,
             an_v, i0_v, i1_v, table_v, rows0_v, rows1_v, so0, so1):
        wid = lax.axis_index("s") * 2 + lax.axis_index("c")
        ebase = wid * epw
        pltpu.sync_copy(table_hbm, table_v)
        pltpu.sync_copy(an_hbm, an_v)
        pltpu.sync_copy(eidx_hbm.at[0, pl.ds(ebase, epw)], i0_v)
        pltpu.sync_copy(eidx_hbm.at[1, pl.ds(ebase, epw)], i1_v)

        lanes = [lax.iota(jnp.int32, _LANES) + g * _LANES
                 for g in range(_C // _LANES)]

        def do_chunk(c, rows_ref, sem):
            off = jnp.minimum(c * _C, epw - _C)
            prs = []
            for g in range(_C // _LANES):
                sl = pl.ds(off + g * _LANES, _LANES)
                a0 = plsc.load_gather(an_v, [i0_v[sl]])
                a1 = plsc.load_gather(an_v, [i1_v[sl]])
                prs.append(a0 * 9 + a1)

            def jbody(j, carry):
                jv = jnp.zeros((_LANES,), jnp.int32) + j
                for g in range(_C // _LANES):
                    vals = plsc.load_gather(table_v, [prs[g], jv])
                    plsc.store_scatter(rows_ref, [lanes[g], jv], vals)
                return carry

            lax.fori_loop(0, _OUT_W, jbody, 0)
            pltpu.async_copy(
                rows_ref, out_hbm.at[pl.ds(ebase + off, _C)], sem)

        def drain(rows_ref, sem):
            pltpu.make_async_copy(
                rows_ref, out_hbm.at[pl.ds(ebase, _C)], sem).wait()

        do_chunk(0, rows0_v, so0)
        do_chunk(1, rows1_v, so1)

        def loop_body(t, carry):
            drain(rows0_v, so0)
            do_chunk(2 * t, rows0_v, so0)
            drain(rows1_v, so1)
            do_chunk(2 * t + 1, rows1_v, so1)
            return carry

        lax.fori_loop(1, nchunks // 2, loop_body, 0)
        drain(rows0_v, so0)
        drain(rows1_v, so1)

    return body


def kernel(atomic_numbers, edge_index, embedding_weight):
    n_atoms = atomic_numbers.shape[0]
    e_total = edge_index.shape[1]
    table = _build_table(embedding_weight)
    info = plsc.get_sparse_core_info()
    num_workers = info.num_cores * info.num_subcores
    if e_total % (num_workers * 8) or e_total // num_workers < _C:
        raise ValueError("unsupported edge count")
    out = _sc_kernel(num_workers, n_atoms, e_total)(
        atomic_numbers, edge_index, table)
    return (out.reshape(e_total, _BASIS, 2 * _CHANNELS), edge_index)


# speedup vs baseline: 2.5728x; 2.5728x over previous
import functools

import jax
import jax.numpy as jnp
from jax import lax
from jax.experimental import pallas as pl
from jax.experimental.pallas import tpu as pltpu
from jax.experimental.pallas import tpu_sc as plsc

_CHANNELS = 16
_SCALAR_MAX = 4
_BASIS = 9
_OUT_W = 2 * _BASIS * _CHANNELS
_NPAIR = 81

_AN_IDX = jnp.array([0, 0, 0, 0, 0, 0, 1, 2, 3], jnp.int32)
_AN_VALID = jnp.array([False, True, False, False, False, False, True, True, True])
_SDIMS = jnp.array([3, 4, 4, 4], jnp.int32)

_C = 128
_LANES = 16


def _build_table(w):
    ia = _AN_IDX[:, None]
    ib = _AN_IDX[None, :]
    valid = _AN_VALID[:, None] & _AN_VALID[None, :]
    sfa = w[ia * 4 + ib].reshape(9, 9, _SCALAR_MAX, _CHANNELS)
    sfb = w[ib * 4 + ia].reshape(9, 9, _SCALAR_MAX, _CHANNELS)
    pad = ((0, 0), (0, 0), (0, _BASIS - _SCALAR_MAX), (0, 0))
    sfa_p = jnp.pad(sfa, pad)
    sfb_p = jnp.pad(sfb, pad)
    rows = jnp.arange(_BASIS)[None, None, :, None]
    ma = valid[:, :, None, None] & (rows < _SDIMS[ia][:, :, None, None])
    mb = valid[:, :, None, None] & (rows < _SDIMS[ib][:, :, None, None])
    ta = jnp.where(ma, sfa_p, 0.0)
    tb = jnp.where(mb, sfb_p, 0.0)
    return jnp.concatenate([ta, tb], axis=-1).reshape(_NPAIR, _OUT_W)


def _sc_kernel(num_workers, n_atoms, e_total):
    epw = e_total // num_workers
    nchunks = pl.cdiv(epw, _C)
    if nchunks % 2 or nchunks < 4:
        raise ValueError("unsupported edge count")
    mesh = plsc.VectorSubcoreMesh(core_axis_name="c", subcore_axis_name="s")

    @functools.partial(
        pl.kernel,
        mesh=mesh,
        compiler_params=pltpu.CompilerParams(
            use_tc_tiling_on_sc=False, needs_layout_passes=False),
        out_type=jax.ShapeDtypeStruct((e_total, _OUT_W), jnp.float32),
        scratch_types=[
            pltpu.VMEM((n_atoms,), jnp.int32),
            pltpu.VMEM((epw,), jnp.int32),
            pltpu.VMEM((epw,), jnp.int32),
            pltpu.VMEM((_NPAIR * _OUT_W,), jnp.float32),
            pltpu.VMEM((_C, _OUT_W), jnp.float32),
            pltpu.VMEM((_C, _OUT_W), jnp.float32),
            pltpu.SemaphoreType.DMA,
            pltpu.SemaphoreType.DMA,
        ],
    )
    def body(an_hbm, eidx_hbm, table_hbm, out_hbm,
             an_v, i0_v, i1_v, table_v, rows0_v, rows1_v, so0, so1):
        wid = lax.axis_index("s") * 2 + lax.axis_index("c")
        ebase = wid * epw
        pltpu.sync_copy(table_hbm, table_v)
        pltpu.sync_copy(an_hbm, an_v)
        pltpu.sync_copy(eidx_hbm.at[0, pl.ds(ebase, epw)], i0_v)
        pltpu.sync_copy(eidx_hbm.at[1, pl.ds(ebase, epw)], i1_v)

        iota = lax.iota(jnp.int32, _LANES)
        _dnums = lax.GatherDimensionNumbers(
            offset_dims=(), collapsed_slice_dims=(0,), start_index_map=(0,))

        def take16(vec, idx):
            return lax.gather(
                vec, idx[:, None], _dnums, slice_sizes=(1,),
                mode=lax.GatherScatterMode.PROMISE_IN_BOUNDS)

        def do_chunk(c, rows_ref, sem):
            off = jnp.minimum(c * _C, epw - _C)
            for g in range(_C // _LANES):
                sl = pl.ds(off + g * _LANES, _LANES)
                a0 = plsc.load_gather(an_v, [i0_v[sl]])
                a1 = plsc.load_gather(an_v, [i1_v[sl]])
                base_g = (a0 * 9 + a1) * _OUT_W

                def ebody(e2, carry):
                    base = take16(
                        base_g, jnp.zeros((_LANES,), jnp.int32) + e2) + iota
                    row = g * _LANES + e2
                    for jb in range(_OUT_W // _LANES):
                        vals = plsc.load_gather(
                            table_v, [base + jb * _LANES])
                        rows_ref[row, pl.ds(jb * _LANES, _LANES)] = vals
                    return carry

                lax.fori_loop(0, _LANES, ebody, 0)
            pltpu.async_copy(
                rows_ref, out_hbm.at[pl.ds(ebase + off, _C)], sem)

        def drain(rows_ref, sem):
            pltpu.make_async_copy(
                rows_ref, out_hbm.at[pl.ds(ebase, _C)], sem).wait()

        do_chunk(0, rows0_v, so0)
        do_chunk(1, rows1_v, so1)

        def loop_body(t, carry):
            drain(rows0_v, so0)
            do_chunk(2 * t, rows0_v, so0)
            drain(rows1_v, so1)
            do_chunk(2 * t + 1, rows1_v, so1)
            return carry

        lax.fori_loop(1, nchunks // 2, loop_body, 0)
        drain(rows0_v, so0)
        drain(rows1_v, so1)

    return body


def kernel(atomic_numbers, edge_index, embedding_weight):
    n_atoms = atomic_numbers.shape[0]
    e_total = edge_index.shape[1]
    table = _build_table(embedding_weight)
    info = plsc.get_sparse_core_info()
    num_workers = info.num_cores * info.num_subcores
    if e_total % (num_workers * 8) or e_total // num_workers < _C:
        raise ValueError("unsupported edge count")
    out = _sc_kernel(num_workers, n_atoms, e_total)(
        atomic_numbers, edge_index, table.reshape(-1))
    return (out.reshape(e_total, _BASIS, 2 * _CHANNELS), edge_index)
